# Initial kernel scaffold; baseline (speedup 1.0000x reference)
import jax
import jax.numpy as jnp
from jax.experimental import pallas as pl


def kernel(edge_index, X, u_Y, W1, b1, W2, b2, g1, be1, g2, be2):
    raise NotImplementedError("write your pallas kernel here")


# speedup vs baseline: 16.3278x; 16.3278x over previous
import functools

import jax
import jax.numpy as jnp
from jax import lax
from jax.experimental import pallas as pl
from jax.experimental.pallas import tpu as pltpu
from jax.experimental.pallas import tpu_sc as plsc

N = 50000
E = 800000
FIN = 96
HID = 128
W = 16
NC1 = FIN // W
NP = 50048
EB = 128
JB = 196
EPT = EB * JB
NTILE = 32
EPAD = EPT * NTILE
RPT = NP // 16
BLK = 2000
NB = N // BLK

_mesh = plsc.VectorSubcoreMesh(core_axis_name="c", subcore_axis_name="s")
_sc_params = pltpu.CompilerParams(use_tc_tiling_on_sc=False)


def _deg_body(dst_hbm, ones_hbm, z_hbm, out_hbm, dst_v, ones_v, acc):
    c = lax.axis_index("c")
    s = lax.axis_index("s")
    wid = c * 16 + s
    pltpu.sync_copy(dst_hbm.at[wid], dst_v)
    pltpu.sync_copy(ones_hbm, ones_v)
    pltpu.sync_copy(z_hbm.at[pl.ds(s * RPT, RPT)], acc.at[pl.ds(s * RPT, RPT)])
    plsc.subcore_barrier()

    @pl.loop(0, JB)
    def _(j):
        pltpu.sync_copy(ones_v, acc.at[dst_v.at[j]], add=True)

    plsc.subcore_barrier()
    pltpu.sync_copy(acc.at[pl.ds(s * RPT, RPT)],
                    out_hbm.at[c].at[pl.ds(s * RPT, RPT)])


@jax.jit
def _deg_call(dstp, ones16, z16):
    f = pl.kernel(
        _deg_body,
        out_type=jax.ShapeDtypeStruct((2, NP, W), jnp.float32),
        mesh=_mesh,
        scratch_types=[
            pltpu.VMEM((JB, EB), jnp.int32),
            pltpu.VMEM((EB, W), jnp.float32),
            pltpu.VMEM_SHARED((NP, W), jnp.float32),
        ],
        compiler_params=_sc_params,
    )
    return f(dstp, ones16, z16)


def _prop_body(nchunk, *refs):
    src_hbm, dst_hbm, z_hbm = refs[:3]
    tabs = refs[3:3 + nchunk]
    outs = refs[3 + nchunk:3 + 2 * nchunk]
    src_v, dst_v, rows0, rows1, acc, gsem0, gsem1 = refs[3 + 2 * nchunk:]

    c = lax.axis_index("c")
    s = lax.axis_index("s")
    wid = c * 16 + s
    pltpu.sync_copy(src_hbm.at[wid], src_v)
    pltpu.sync_copy(dst_hbm.at[wid], dst_v)

    for k in range(nchunk):
        tab = tabs[k]

        @pl.when(c == 0)
        def _():
            pltpu.sync_copy(tab.at[pl.ds(s * RPT, RPT)],
                            acc.at[pl.ds(s * RPT, RPT)])

        @pl.when(c == 1)
        def _():
            pltpu.sync_copy(z_hbm.at[pl.ds(s * RPT, RPT)],
                            acc.at[pl.ds(s * RPT, RPT)])

        plsc.subcore_barrier()

        pltpu.async_copy(tab.at[src_v.at[0]], rows0, gsem0)

        @pl.loop(0, JB, step=2)
        def _(j):
            pltpu.async_copy(tab.at[src_v.at[j + 1]], rows1, gsem1)
            pltpu.make_async_copy(tab.at[src_v.at[j]], rows0, gsem0).wait()
            pltpu.sync_copy(rows0, acc.at[dst_v.at[j]], add=True)

            @pl.when(j + 2 < JB)
            def _():
                pltpu.async_copy(tab.at[src_v.at[j + 2]], rows0, gsem0)

            pltpu.make_async_copy(tab.at[src_v.at[j + 1]], rows1, gsem1).wait()
            pltpu.sync_copy(rows1, acc.at[dst_v.at[j + 1]], add=True)

        plsc.subcore_barrier()
        pltpu.sync_copy(acc.at[pl.ds(s * RPT, RPT)],
                        outs[k].at[c].at[pl.ds(s * RPT, RPT)])
        if k + 1 < nchunk:
            plsc.subcore_barrier()


def _prop_call(nchunk, srcp, dstp, z16, *tabs):
    f = pl.kernel(
        functools.partial(_prop_body, nchunk),
        out_type=[jax.ShapeDtypeStruct((2, NP, W), jnp.float32)] * nchunk,
        mesh=_mesh,
        scratch_types=[
            pltpu.VMEM((JB, EB), jnp.int32),
            pltpu.VMEM((JB, EB), jnp.int32),
            pltpu.VMEM((EB, W), jnp.float32),
            pltpu.VMEM((EB, W), jnp.float32),
            pltpu.VMEM_SHARED((NP, W), jnp.float32),
            pltpu.SemaphoreType.DMA,
            pltpu.SemaphoreType.DMA,
        ],
        compiler_params=_sc_params,
    )
    return f(srcp, dstp, z16, *tabs)


_prop6_call = jax.jit(functools.partial(_prop_call, NC1))
_prop1_call = jax.jit(functools.partial(_prop_call, 1))


def _t1_body(deg_ref, l_ref, dinv_ref, *lc_refs):
    d = deg_ref[0, :, 0:1] + deg_ref[1, :, 0:1] + 1.0
    dinv = lax.rsqrt(d)
    dinv_ref[...] = dinv
    lp = l_ref[...] * dinv
    for k in range(NC1):
        lc_refs[k][...] = lp[:, k * W:(k + 1) * W]


@jax.jit
def _t1_call(deg, L):
    return pl.pallas_call(
        _t1_body,
        grid=(NB,),
        in_specs=[
            pl.BlockSpec((2, BLK, W), lambda i: (0, i, 0)),
            pl.BlockSpec((BLK, FIN), lambda i: (i, 0)),
        ],
        out_specs=[pl.BlockSpec((BLK, 1), lambda i: (i, 0))] +
                  [pl.BlockSpec((BLK, W), lambda i: (i, 0))] * NC1,
        out_shape=[jax.ShapeDtypeStruct((N, 1), jnp.float32)] +
                  [jax.ShapeDtypeStruct((NP, W), jnp.float32)] * NC1,
    )(deg, L)


def _ta_body(*refs):
    accs = refs[:NC1]
    dinv, w1, b1, h_ref, st_ref = refs[NC1:]
    dv = dinv[...]
    z = jnp.concatenate(
        [(accs[k][0] + accs[k][1]) * dv for k in range(NC1)], axis=1)
    h = jnp.dot(z, w1[...], preferred_element_type=jnp.float32) + b1[...]
    h_ref[...] = h

    @pl.when(pl.program_id(0) == 0)
    def _():
        st_ref[...] = jnp.zeros_like(st_ref)

    st_ref[0:1, :] += jnp.sum(h, axis=0, keepdims=True)
    st_ref[1:2, :] += jnp.sum(h * h, axis=0, keepdims=True)


@jax.jit
def _ta_call(accs, dinv, W1, b1):
    acc_spec = pl.BlockSpec((2, BLK, W), lambda i: (0, i, 0))
    return pl.pallas_call(
        _ta_body,
        grid=(NB,),
        in_specs=[acc_spec] * NC1 + [
            pl.BlockSpec((BLK, 1), lambda i: (i, 0)),
            pl.BlockSpec((FIN, HID), lambda i: (0, 0)),
            pl.BlockSpec((1, HID), lambda i: (0, 0)),
        ],
        out_specs=[
            pl.BlockSpec((BLK, HID), lambda i: (i, 0)),
            pl.BlockSpec((8, HID), lambda i: (0, 0)),
        ],
        out_shape=[
            jax.ShapeDtypeStruct((N, HID), jnp.float32),
            jax.ShapeDtypeStruct((8, HID), jnp.float32),
        ],
    )(*accs, dinv, W1, b1)


def _tb_body(h, st, g1, be1, dinv, w2p, gp_ref):
    inv_n = 1.0 / N
    m = st[0:1, :] * inv_n
    v = st[1:2, :] * inv_n - m * m
    y = jnp.maximum((h[...] - m) * lax.rsqrt(v + 1e-5) * g1[...] + be1[...],
                    0.0)
    g = jnp.dot(y, w2p[...], preferred_element_type=jnp.float32)
    gp_ref[...] = g * dinv[...]


@jax.jit
def _tb_call(H, st1, g1, be1, dinv, W2p):
    return pl.pallas_call(
        _tb_body,
        grid=(NB,),
        in_specs=[
            pl.BlockSpec((BLK, HID), lambda i: (i, 0)),
            pl.BlockSpec((8, HID), lambda i: (0, 0)),
            pl.BlockSpec((1, HID), lambda i: (0, 0)),
            pl.BlockSpec((1, HID), lambda i: (0, 0)),
            pl.BlockSpec((BLK, 1), lambda i: (i, 0)),
            pl.BlockSpec((HID, W), lambda i: (0, 0)),
        ],
        out_specs=pl.BlockSpec((BLK, W), lambda i: (i, 0)),
        out_shape=jax.ShapeDtypeStruct((NP, W), jnp.float32),
    )(H, st1, g1, be1, dinv, W2p)


def _tc1_body(a2, dinv, b2p, s_ref, st_ref):
    s = (a2[0] + a2[1]) * dinv[...] + b2p[...]
    s_ref[...] = s

    @pl.when(pl.program_id(0) == 0)
    def _():
        st_ref[...] = jnp.zeros_like(st_ref)

    st_ref[0:1, :] += jnp.sum(s, axis=0, keepdims=True)
    st_ref[1:2, :] += jnp.sum(s * s, axis=0, keepdims=True)


@jax.jit
def _tc1_call(a2, dinv, b2p):
    return pl.pallas_call(
        _tc1_body,
        grid=(NB,),
        in_specs=[
            pl.BlockSpec((2, BLK, W), lambda i: (0, i, 0)),
            pl.BlockSpec((BLK, 1), lambda i: (i, 0)),
            pl.BlockSpec((1, W), lambda i: (0, 0)),
        ],
        out_specs=[
            pl.BlockSpec((BLK, W), lambda i: (i, 0)),
            pl.BlockSpec((8, W), lambda i: (0, 0)),
        ],
        out_shape=[
            jax.ShapeDtypeStruct((N, W), jnp.float32),
            jax.ShapeDtypeStruct((8, W), jnp.float32),
        ],
    )(a2, dinv, b2p)


def _tc2_body(s, st, g2p, be2p, out_ref):
    inv_n = 1.0 / N
    m = st[0:1, :] * inv_n
    v = st[1:2, :] * inv_n - m * m
    b = (s[...] - m) * lax.rsqrt(v + 1e-5) * g2p[...] + be2p[...]
    t = b[:, 0:2]
    mx = jnp.max(t, axis=1, keepdims=True)
    e = jnp.exp(t - mx)
    out_ref[...] = e / jnp.sum(e, axis=1, keepdims=True)


@jax.jit
def _tc2_call(S, st2, g2p, be2p):
    return pl.pallas_call(
        _tc2_body,
        grid=(NB,),
        in_specs=[
            pl.BlockSpec((BLK, W), lambda i: (i, 0)),
            pl.BlockSpec((8, W), lambda i: (0, 0)),
            pl.BlockSpec((1, W), lambda i: (0, 0)),
            pl.BlockSpec((1, W), lambda i: (0, 0)),
        ],
        out_specs=pl.BlockSpec((BLK, 2), lambda i: (i, 0)),
        out_shape=jax.ShapeDtypeStruct((N, 2), jnp.float32),
    )(S, st2, g2p, be2p)


def kernel(edge_index, X, u_Y, W1, b1, W2, b2, g1, be1, g2, be2):
    ei = edge_index.astype(jnp.int32)
    pad = EPAD - E
    srcp = jnp.concatenate([ei[0], jnp.zeros((pad,), jnp.int32)])
    srcp = srcp.reshape(NTILE, JB, EB)
    dstp = jnp.concatenate([ei[1], jnp.full((pad,), N, jnp.int32)])
    dstp = dstp.reshape(NTILE, JB, EB)

    ones16 = jnp.ones((EB, W), jnp.float32)
    z16 = jnp.zeros((NP, W), jnp.float32)

    deg = _deg_call(dstp, ones16, z16)

    L = jnp.concatenate([u_Y, X], axis=1)
    t1 = _t1_call(deg, L)
    dinv, lcs = t1[0], t1[1:]

    accs = _prop6_call(srcp, dstp, z16, *lcs)

    H, st1 = _ta_call(accs, dinv, W1, b1.reshape(1, HID))

    W2p = jnp.pad(W2, ((0, 0), (0, W - 2)))
    Gp = _tb_call(H, st1, g1.reshape(1, HID), be1.reshape(1, HID), dinv, W2p)

    (a20,) = _prop1_call(srcp, dstp, z16, Gp)

    b2p = jnp.pad(b2, (0, W - 2)).reshape(1, W)
    g2p = jnp.pad(g2, (0, W - 2)).reshape(1, W)
    be2p = jnp.pad(be2, (0, W - 2)).reshape(1, W)
    S, st2 = _tc1_call(a20, dinv, b2p)
    return _tc2_call(S, st2, g2p, be2p)


# speedup vs baseline: 18.9251x; 1.1591x over previous
import functools

import jax
import jax.numpy as jnp
from jax import lax
from jax.experimental import pallas as pl
from jax.experimental.pallas import tpu as pltpu
from jax.experimental.pallas import tpu_sc as plsc

N = 50000
E = 800000
FIN = 96
HID = 128
W = 16
NC1 = FIN // W
NP = 50048
EB = 128
JB = 196
EPT = EB * JB
NTILE = 32
EPAD = EPT * NTILE
RPT = NP // 16
BLK = 2000
NB = N // BLK

_mesh = plsc.VectorSubcoreMesh(core_axis_name="c", subcore_axis_name="s")
_sc_params = pltpu.CompilerParams(use_tc_tiling_on_sc=False)


def _deg_body(dst_hbm, ones_hbm, z_hbm, out_hbm, dst_v, ones_v, acc):
    c = lax.axis_index("c")
    s = lax.axis_index("s")
    wid = c * 16 + s
    pltpu.sync_copy(dst_hbm.at[wid], dst_v)
    pltpu.sync_copy(ones_hbm, ones_v)
    pltpu.sync_copy(z_hbm.at[pl.ds(s * RPT, RPT)], acc.at[pl.ds(s * RPT, RPT)])
    plsc.subcore_barrier()

    @pl.loop(0, JB)
    def _(j):
        pltpu.sync_copy(ones_v, acc.at[dst_v.at[j]], add=True)

    plsc.subcore_barrier()
    pltpu.sync_copy(acc.at[pl.ds(s * RPT, RPT)],
                    out_hbm.at[c].at[pl.ds(s * RPT, RPT)])


@jax.jit
def _deg_call(dstp, ones16, z16):
    f = pl.kernel(
        _deg_body,
        out_type=jax.ShapeDtypeStruct((2, NP, W), jnp.float32),
        mesh=_mesh,
        scratch_types=[
            pltpu.VMEM((JB, EB), jnp.int32),
            pltpu.VMEM((EB, W), jnp.float32),
            pltpu.VMEM_SHARED((NP, W), jnp.float32),
        ],
        compiler_params=_sc_params,
    )
    return f(dstp, ones16, z16)


def _prop_body(nchunk, *refs):
    src_hbm, dst_hbm, z_hbm = refs[:3]
    tabs = refs[3:3 + nchunk]
    outs = refs[3 + nchunk:3 + 2 * nchunk]
    rest = refs[3 + 2 * nchunk:]
    src_v, dst_v = rest[0], rest[1]
    rows = rest[2:6]
    acc = rest[6]
    gsems = rest[7:11]
    ssems = rest[11:15]

    c = lax.axis_index("c")
    s = lax.axis_index("s")
    wid = c * 16 + s
    pltpu.sync_copy(src_hbm.at[wid], src_v)
    pltpu.sync_copy(dst_hbm.at[wid], dst_v)

    for k in range(nchunk):
        tab = tabs[k]

        @pl.when(c == 0)
        def _():
            pltpu.sync_copy(tab.at[pl.ds(s * RPT, RPT)],
                            acc.at[pl.ds(s * RPT, RPT)])

        @pl.when(c == 1)
        def _():
            pltpu.sync_copy(z_hbm.at[pl.ds(s * RPT, RPT)],
                            acc.at[pl.ds(s * RPT, RPT)])

        plsc.subcore_barrier()

        for b in range(4):
            pltpu.async_copy(tab.at[src_v.at[b]], rows[b], gsems[b])

        @pl.loop(0, JB, step=4)
        def _(j):
            for b in range(4):
                pltpu.make_async_copy(
                    tab.at[src_v.at[j + b]], rows[b], gsems[b]).wait()
                pltpu.async_copy(
                    rows[b], acc.at[dst_v.at[j + b]], ssems[b], add=True)
            for b in range(4):
                @pl.when(j + b + 4 < JB)
                def _():
                    pltpu.make_async_copy(
                        rows[b], acc.at[dst_v.at[j + b]], ssems[b]).wait()
                    pltpu.async_copy(
                        tab.at[src_v.at[j + b + 4]], rows[b], gsems[b])

        for b in range(4):
            pltpu.make_async_copy(
                rows[b], acc.at[dst_v.at[JB - 4 + b]], ssems[b]).wait()

        plsc.subcore_barrier()
        pltpu.sync_copy(acc.at[pl.ds(s * RPT, RPT)],
                        outs[k].at[c].at[pl.ds(s * RPT, RPT)])
        if k + 1 < nchunk:
            plsc.subcore_barrier()


def _prop_call(nchunk, srcp, dstp, z16, *tabs):
    f = pl.kernel(
        functools.partial(_prop_body, nchunk),
        out_type=[jax.ShapeDtypeStruct((2, NP, W), jnp.float32)] * nchunk,
        mesh=_mesh,
        scratch_types=[
            pltpu.VMEM((JB, EB), jnp.int32),
            pltpu.VMEM((JB, EB), jnp.int32),
        ] + [pltpu.VMEM((EB, W), jnp.float32)] * 4 + [
            pltpu.VMEM_SHARED((NP, W), jnp.float32),
        ] + [pltpu.SemaphoreType.DMA] * 8,
        compiler_params=_sc_params,
    )
    return f(srcp, dstp, z16, *tabs)


_prop6_call = jax.jit(functools.partial(_prop_call, NC1))
_prop1_call = jax.jit(functools.partial(_prop_call, 1))


def _t1_body(deg_ref, l_ref, dinv_ref, *lc_refs):
    d = deg_ref[0, :, 0:1] + deg_ref[1, :, 0:1] + 1.0
    dinv = lax.rsqrt(d)
    dinv_ref[...] = dinv
    lp = l_ref[...] * dinv
    for k in range(NC1):
        lc_refs[k][...] = lp[:, k * W:(k + 1) * W]


@jax.jit
def _t1_call(deg, L):
    return pl.pallas_call(
        _t1_body,
        grid=(NB,),
        in_specs=[
            pl.BlockSpec((2, BLK, W), lambda i: (0, i, 0)),
            pl.BlockSpec((BLK, FIN), lambda i: (i, 0)),
        ],
        out_specs=[pl.BlockSpec((BLK, 1), lambda i: (i, 0))] +
                  [pl.BlockSpec((BLK, W), lambda i: (i, 0))] * NC1,
        out_shape=[jax.ShapeDtypeStruct((N, 1), jnp.float32)] +
                  [jax.ShapeDtypeStruct((NP, W), jnp.float32)] * NC1,
    )(deg, L)


def _ta_body(*refs):
    accs = refs[:NC1]
    dinv, w1, b1, h_ref, st_ref = refs[NC1:]
    dv = dinv[...]
    z = jnp.concatenate(
        [(accs[k][0] + accs[k][1]) * dv for k in range(NC1)], axis=1)
    h = jnp.dot(z, w1[...], preferred_element_type=jnp.float32) + b1[...]
    h_ref[...] = h

    @pl.when(pl.program_id(0) == 0)
    def _():
        st_ref[...] = jnp.zeros_like(st_ref)

    st_ref[0:1, :] += jnp.sum(h, axis=0, keepdims=True)
    st_ref[1:2, :] += jnp.sum(h * h, axis=0, keepdims=True)


@jax.jit
def _ta_call(accs, dinv, W1, b1):
    acc_spec = pl.BlockSpec((2, BLK, W), lambda i: (0, i, 0))
    return pl.pallas_call(
        _ta_body,
        grid=(NB,),
        in_specs=[acc_spec] * NC1 + [
            pl.BlockSpec((BLK, 1), lambda i: (i, 0)),
            pl.BlockSpec((FIN, HID), lambda i: (0, 0)),
            pl.BlockSpec((1, HID), lambda i: (0, 0)),
        ],
        out_specs=[
            pl.BlockSpec((BLK, HID), lambda i: (i, 0)),
            pl.BlockSpec((8, HID), lambda i: (0, 0)),
        ],
        out_shape=[
            jax.ShapeDtypeStruct((N, HID), jnp.float32),
            jax.ShapeDtypeStruct((8, HID), jnp.float32),
        ],
    )(*accs, dinv, W1, b1)


def _tb_body(h, st, g1, be1, dinv, w2p, gp_ref):
    inv_n = 1.0 / N
    m = st[0:1, :] * inv_n
    v = st[1:2, :] * inv_n - m * m
    y = jnp.maximum((h[...] - m) * lax.rsqrt(v + 1e-5) * g1[...] + be1[...],
                    0.0)
    g = jnp.dot(y, w2p[...], preferred_element_type=jnp.float32)
    gp_ref[...] = g * dinv[...]


@jax.jit
def _tb_call(H, st1, g1, be1, dinv, W2p):
    return pl.pallas_call(
        _tb_body,
        grid=(NB,),
        in_specs=[
            pl.BlockSpec((BLK, HID), lambda i: (i, 0)),
            pl.BlockSpec((8, HID), lambda i: (0, 0)),
            pl.BlockSpec((1, HID), lambda i: (0, 0)),
            pl.BlockSpec((1, HID), lambda i: (0, 0)),
            pl.BlockSpec((BLK, 1), lambda i: (i, 0)),
            pl.BlockSpec((HID, W), lambda i: (0, 0)),
        ],
        out_specs=pl.BlockSpec((BLK, W), lambda i: (i, 0)),
        out_shape=jax.ShapeDtypeStruct((NP, W), jnp.float32),
    )(H, st1, g1, be1, dinv, W2p)


def _tc1_body(a2, dinv, b2p, s_ref, st_ref):
    s = (a2[0] + a2[1]) * dinv[...] + b2p[...]
    s_ref[...] = s

    @pl.when(pl.program_id(0) == 0)
    def _():
        st_ref[...] = jnp.zeros_like(st_ref)

    st_ref[0:1, :] += jnp.sum(s, axis=0, keepdims=True)
    st_ref[1:2, :] += jnp.sum(s * s, axis=0, keepdims=True)


@jax.jit
def _tc1_call(a2, dinv, b2p):
    return pl.pallas_call(
        _tc1_body,
        grid=(NB,),
        in_specs=[
            pl.BlockSpec((2, BLK, W), lambda i: (0, i, 0)),
            pl.BlockSpec((BLK, 1), lambda i: (i, 0)),
            pl.BlockSpec((1, W), lambda i: (0, 0)),
        ],
        out_specs=[
            pl.BlockSpec((BLK, W), lambda i: (i, 0)),
            pl.BlockSpec((8, W), lambda i: (0, 0)),
        ],
        out_shape=[
            jax.ShapeDtypeStruct((N, W), jnp.float32),
            jax.ShapeDtypeStruct((8, W), jnp.float32),
        ],
    )(a2, dinv, b2p)


def _tc2_body(s, st, g2p, be2p, out_ref):
    inv_n = 1.0 / N
    m = st[0:1, :] * inv_n
    v = st[1:2, :] * inv_n - m * m
    b = (s[...] - m) * lax.rsqrt(v + 1e-5) * g2p[...] + be2p[...]
    t = b[:, 0:2]
    mx = jnp.max(t, axis=1, keepdims=True)
    e = jnp.exp(t - mx)
    out_ref[...] = e / jnp.sum(e, axis=1, keepdims=True)


@jax.jit
def _tc2_call(S, st2, g2p, be2p):
    return pl.pallas_call(
        _tc2_body,
        grid=(NB,),
        in_specs=[
            pl.BlockSpec((BLK, W), lambda i: (i, 0)),
            pl.BlockSpec((8, W), lambda i: (0, 0)),
            pl.BlockSpec((1, W), lambda i: (0, 0)),
            pl.BlockSpec((1, W), lambda i: (0, 0)),
        ],
        out_specs=pl.BlockSpec((BLK, 2), lambda i: (i, 0)),
        out_shape=jax.ShapeDtypeStruct((N, 2), jnp.float32),
    )(S, st2, g2p, be2p)


def kernel(edge_index, X, u_Y, W1, b1, W2, b2, g1, be1, g2, be2):
    ei = edge_index.astype(jnp.int32)
    pad = EPAD - E
    srcp = jnp.concatenate([ei[0], jnp.zeros((pad,), jnp.int32)])
    srcp = srcp.reshape(NTILE, JB, EB)
    dstp = jnp.concatenate([ei[1], jnp.full((pad,), N, jnp.int32)])
    dstp = dstp.reshape(NTILE, JB, EB)

    ones16 = jnp.ones((EB, W), jnp.float32)
    z16 = jnp.zeros((NP, W), jnp.float32)

    deg = _deg_call(dstp, ones16, z16)

    L = jnp.concatenate([u_Y, X], axis=1)
    t1 = _t1_call(deg, L)
    dinv, lcs = t1[0], t1[1:]

    accs = _prop6_call(srcp, dstp, z16, *lcs)

    H, st1 = _ta_call(accs, dinv, W1, b1.reshape(1, HID))

    W2p = jnp.pad(W2, ((0, 0), (0, W - 2)))
    Gp = _tb_call(H, st1, g1.reshape(1, HID), be1.reshape(1, HID), dinv, W2p)

    (a20,) = _prop1_call(srcp, dstp, z16, Gp)

    b2p = jnp.pad(b2, (0, W - 2)).reshape(1, W)
    g2p = jnp.pad(g2, (0, W - 2)).reshape(1, W)
    be2p = jnp.pad(be2, (0, W - 2)).reshape(1, W)
    S, st2 = _tc1_call(a20, dinv, b2p)
    return _tc2_call(S, st2, g2p, be2p)


# speedup vs baseline: 19.0604x; 1.0071x over previous
import functools

import jax
import jax.numpy as jnp
from jax import lax
from jax.experimental import pallas as pl
from jax.experimental.pallas import tpu as pltpu
from jax.experimental.pallas import tpu_sc as plsc

N = 50000
E = 800000
FIN = 96
HID = 128
W = 16
NC1 = FIN // W
NP = 50048
EB = 128
JB = 196
EPT = EB * JB
NTILE = 32
EPAD = EPT * NTILE
RPT = NP // 16
BLK = 2000
NB = N // BLK

_mesh = plsc.VectorSubcoreMesh(core_axis_name="c", subcore_axis_name="s")
_sc_params = pltpu.CompilerParams(use_tc_tiling_on_sc=False)


def _deg_body(dst_hbm, ones_hbm, z_hbm, out_hbm, dst_v, ones_v, acc, *ssems):
    c = lax.axis_index("c")
    s = lax.axis_index("s")
    wid = c * 16 + s
    pltpu.sync_copy(dst_hbm.at[wid], dst_v)
    pltpu.sync_copy(ones_hbm, ones_v)
    pltpu.sync_copy(z_hbm.at[pl.ds(s * RPT, RPT)], acc.at[pl.ds(s * RPT, RPT)])
    plsc.subcore_barrier()

    for b in range(4):
        pltpu.async_copy(ones_v, acc.at[dst_v.at[b]], ssems[b], add=True)

    @pl.loop(4, JB, step=4)
    def _(j):
        for b in range(4):
            pltpu.make_async_copy(
                ones_v, acc.at[dst_v.at[j - 4 + b]], ssems[b]).wait()
            pltpu.async_copy(ones_v, acc.at[dst_v.at[j + b]], ssems[b],
                             add=True)

    for b in range(4):
        pltpu.make_async_copy(
            ones_v, acc.at[dst_v.at[JB - 4 + b]], ssems[b]).wait()

    plsc.subcore_barrier()
    pltpu.sync_copy(acc.at[pl.ds(s * RPT, RPT)],
                    out_hbm.at[c].at[pl.ds(s * RPT, RPT)])


@jax.jit
def _deg_call(dstp, ones16, z16):
    f = pl.kernel(
        _deg_body,
        out_type=jax.ShapeDtypeStruct((2, NP, W), jnp.float32),
        mesh=_mesh,
        scratch_types=[
            pltpu.VMEM((JB, EB), jnp.int32),
            pltpu.VMEM((EB, W), jnp.float32),
            pltpu.VMEM_SHARED((NP, W), jnp.float32),
        ] + [pltpu.SemaphoreType.DMA] * 4,
        compiler_params=_sc_params,
    )
    return f(dstp, ones16, z16)


def _prop_body(nchunk, *refs):
    src_hbm, dst_hbm, z_hbm = refs[:3]
    tabs = refs[3:3 + nchunk]
    outs = refs[3 + nchunk:3 + 2 * nchunk]
    rest = refs[3 + 2 * nchunk:]
    src_v, dst_v = rest[0], rest[1]
    rows = rest[2:6]
    acc = rest[6]
    gsems = rest[7:11]
    ssems = rest[11:15]

    c = lax.axis_index("c")
    s = lax.axis_index("s")
    wid = c * 16 + s
    pltpu.sync_copy(src_hbm.at[wid], src_v)
    pltpu.sync_copy(dst_hbm.at[wid], dst_v)

    for k in range(nchunk):
        tab = tabs[k]

        @pl.when(c == 0)
        def _():
            pltpu.sync_copy(tab.at[pl.ds(s * RPT, RPT)],
                            acc.at[pl.ds(s * RPT, RPT)])

        @pl.when(c == 1)
        def _():
            pltpu.sync_copy(z_hbm.at[pl.ds(s * RPT, RPT)],
                            acc.at[pl.ds(s * RPT, RPT)])

        plsc.subcore_barrier()

        for b in range(4):
            pltpu.async_copy(tab.at[src_v.at[b]], rows[b], gsems[b])

        @pl.loop(0, JB, step=4)
        def _(j):
            for b in range(4):
                pltpu.make_async_copy(
                    tab.at[src_v.at[j + b]], rows[b], gsems[b]).wait()
                pltpu.async_copy(
                    rows[b], acc.at[dst_v.at[j + b]], ssems[b], add=True)
            for b in range(4):
                @pl.when(j + b + 4 < JB)
                def _():
                    pltpu.make_async_copy(
                        rows[b], acc.at[dst_v.at[j + b]], ssems[b]).wait()
                    pltpu.async_copy(
                        tab.at[src_v.at[j + b + 4]], rows[b], gsems[b])

        for b in range(4):
            pltpu.make_async_copy(
                rows[b], acc.at[dst_v.at[JB - 4 + b]], ssems[b]).wait()

        plsc.subcore_barrier()
        pltpu.sync_copy(acc.at[pl.ds(s * RPT, RPT)],
                        outs[k].at[c].at[pl.ds(s * RPT, RPT)])
        if k + 1 < nchunk:
            plsc.subcore_barrier()


def _prop_call(nchunk, srcp, dstp, z16, *tabs):
    f = pl.kernel(
        functools.partial(_prop_body, nchunk),
        out_type=[jax.ShapeDtypeStruct((2, NP, W), jnp.float32)] * nchunk,
        mesh=_mesh,
        scratch_types=[
            pltpu.VMEM((JB, EB), jnp.int32),
            pltpu.VMEM((JB, EB), jnp.int32),
        ] + [pltpu.VMEM((EB, W), jnp.float32)] * 4 + [
            pltpu.VMEM_SHARED((NP, W), jnp.float32),
        ] + [pltpu.SemaphoreType.DMA] * 8,
        compiler_params=_sc_params,
    )
    return f(srcp, dstp, z16, *tabs)


_prop6_call = jax.jit(functools.partial(_prop_call, NC1))
_prop1_call = jax.jit(functools.partial(_prop_call, 1))


def _t1_body(deg_ref, l_ref, dinv_ref, *lc_refs):
    d = deg_ref[0, :, 0:1] + deg_ref[1, :, 0:1] + 1.0
    dinv = lax.rsqrt(d)
    dinv_ref[...] = dinv
    lp = l_ref[...] * dinv
    for k in range(NC1):
        lc_refs[k][...] = lp[:, k * W:(k + 1) * W]


@jax.jit
def _t1_call(deg, L):
    return pl.pallas_call(
        _t1_body,
        grid=(NB,),
        in_specs=[
            pl.BlockSpec((2, BLK, W), lambda i: (0, i, 0)),
            pl.BlockSpec((BLK, FIN), lambda i: (i, 0)),
        ],
        out_specs=[pl.BlockSpec((BLK, 1), lambda i: (i, 0))] +
                  [pl.BlockSpec((BLK, W), lambda i: (i, 0))] * NC1,
        out_shape=[jax.ShapeDtypeStruct((N, 1), jnp.float32)] +
                  [jax.ShapeDtypeStruct((NP, W), jnp.float32)] * NC1,
    )(deg, L)


def _ta_body(*refs):
    accs = refs[:NC1]
    dinv, w1, b1, h_ref, st_ref = refs[NC1:]
    dv = dinv[...]
    z = jnp.concatenate(
        [(accs[k][0] + accs[k][1]) * dv for k in range(NC1)], axis=1)
    h = jnp.dot(z, w1[...], preferred_element_type=jnp.float32) + b1[...]
    h_ref[...] = h

    @pl.when(pl.program_id(0) == 0)
    def _():
        st_ref[...] = jnp.zeros_like(st_ref)

    st_ref[0:1, :] += jnp.sum(h, axis=0, keepdims=True)
    st_ref[1:2, :] += jnp.sum(h * h, axis=0, keepdims=True)


@jax.jit
def _ta_call(accs, dinv, W1, b1):
    acc_spec = pl.BlockSpec((2, BLK, W), lambda i: (0, i, 0))
    return pl.pallas_call(
        _ta_body,
        grid=(NB,),
        in_specs=[acc_spec] * NC1 + [
            pl.BlockSpec((BLK, 1), lambda i: (i, 0)),
            pl.BlockSpec((FIN, HID), lambda i: (0, 0)),
            pl.BlockSpec((1, HID), lambda i: (0, 0)),
        ],
        out_specs=[
            pl.BlockSpec((BLK, HID), lambda i: (i, 0)),
            pl.BlockSpec((8, HID), lambda i: (0, 0)),
        ],
        out_shape=[
            jax.ShapeDtypeStruct((N, HID), jnp.float32),
            jax.ShapeDtypeStruct((8, HID), jnp.float32),
        ],
    )(*accs, dinv, W1, b1)


def _tb_body(h, st, g1, be1, dinv, w2p, gp_ref):
    inv_n = 1.0 / N
    m = st[0:1, :] * inv_n
    v = st[1:2, :] * inv_n - m * m
    y = jnp.maximum((h[...] - m) * lax.rsqrt(v + 1e-5) * g1[...] + be1[...],
                    0.0)
    g = jnp.dot(y, w2p[...], preferred_element_type=jnp.float32)
    gp_ref[...] = g * dinv[...]


@jax.jit
def _tb_call(H, st1, g1, be1, dinv, W2p):
    return pl.pallas_call(
        _tb_body,
        grid=(NB,),
        in_specs=[
            pl.BlockSpec((BLK, HID), lambda i: (i, 0)),
            pl.BlockSpec((8, HID), lambda i: (0, 0)),
            pl.BlockSpec((1, HID), lambda i: (0, 0)),
            pl.BlockSpec((1, HID), lambda i: (0, 0)),
            pl.BlockSpec((BLK, 1), lambda i: (i, 0)),
            pl.BlockSpec((HID, W), lambda i: (0, 0)),
        ],
        out_specs=pl.BlockSpec((BLK, W), lambda i: (i, 0)),
        out_shape=jax.ShapeDtypeStruct((NP, W), jnp.float32),
    )(H, st1, g1, be1, dinv, W2p)


def _tc1_body(a2, dinv, b2p, s_ref, st_ref):
    s = (a2[0] + a2[1]) * dinv[...] + b2p[...]
    s_ref[...] = s

    @pl.when(pl.program_id(0) == 0)
    def _():
        st_ref[...] = jnp.zeros_like(st_ref)

    st_ref[0:1, :] += jnp.sum(s, axis=0, keepdims=True)
    st_ref[1:2, :] += jnp.sum(s * s, axis=0, keepdims=True)


@jax.jit
def _tc1_call(a2, dinv, b2p):
    return pl.pallas_call(
        _tc1_body,
        grid=(NB,),
        in_specs=[
            pl.BlockSpec((2, BLK, W), lambda i: (0, i, 0)),
            pl.BlockSpec((BLK, 1), lambda i: (i, 0)),
            pl.BlockSpec((1, W), lambda i: (0, 0)),
        ],
        out_specs=[
            pl.BlockSpec((BLK, W), lambda i: (i, 0)),
            pl.BlockSpec((8, W), lambda i: (0, 0)),
        ],
        out_shape=[
            jax.ShapeDtypeStruct((N, W), jnp.float32),
            jax.ShapeDtypeStruct((8, W), jnp.float32),
        ],
    )(a2, dinv, b2p)


def _tc2_body(s, st, g2p, be2p, out_ref):
    inv_n = 1.0 / N
    m = st[0:1, :] * inv_n
    v = st[1:2, :] * inv_n - m * m
    b = (s[...] - m) * lax.rsqrt(v + 1e-5) * g2p[...] + be2p[...]
    t = b[:, 0:2]
    mx = jnp.max(t, axis=1, keepdims=True)
    e = jnp.exp(t - mx)
    out_ref[...] = e / jnp.sum(e, axis=1, keepdims=True)


@jax.jit
def _tc2_call(S, st2, g2p, be2p):
    return pl.pallas_call(
        _tc2_body,
        grid=(NB,),
        in_specs=[
            pl.BlockSpec((BLK, W), lambda i: (i, 0)),
            pl.BlockSpec((8, W), lambda i: (0, 0)),
            pl.BlockSpec((1, W), lambda i: (0, 0)),
            pl.BlockSpec((1, W), lambda i: (0, 0)),
        ],
        out_specs=pl.BlockSpec((BLK, 2), lambda i: (i, 0)),
        out_shape=jax.ShapeDtypeStruct((N, 2), jnp.float32),
    )(S, st2, g2p, be2p)


def kernel(edge_index, X, u_Y, W1, b1, W2, b2, g1, be1, g2, be2):
    ei = edge_index.astype(jnp.int32)
    pad = EPAD - E
    srcp = jnp.concatenate([ei[0], jnp.zeros((pad,), jnp.int32)])
    srcp = srcp.reshape(NTILE, JB, EB)
    dstp = jnp.concatenate([ei[1], jnp.full((pad,), N, jnp.int32)])
    dstp = dstp.reshape(NTILE, JB, EB)

    ones16 = jnp.ones((EB, W), jnp.float32)
    z16 = jnp.zeros((NP, W), jnp.float32)

    deg = _deg_call(dstp, ones16, z16)

    L = jnp.concatenate([u_Y, X], axis=1)
    t1 = _t1_call(deg, L)
    dinv, lcs = t1[0], t1[1:]

    accs = _prop6_call(srcp, dstp, z16, *lcs)

    H, st1 = _ta_call(accs, dinv, W1, b1.reshape(1, HID))

    W2p = jnp.pad(W2, ((0, 0), (0, W - 2)))
    Gp = _tb_call(H, st1, g1.reshape(1, HID), be1.reshape(1, HID), dinv, W2p)

    (a20,) = _prop1_call(srcp, dstp, z16, Gp)

    b2p = jnp.pad(b2, (0, W - 2)).reshape(1, W)
    g2p = jnp.pad(g2, (0, W - 2)).reshape(1, W)
    be2p = jnp.pad(be2, (0, W - 2)).reshape(1, W)
    S, st2 = _tc1_call(a20, dinv, b2p)
    return _tc2_call(S, st2, g2p, be2p)
